# baseline (speedup 1.0000x reference)
import functools

import jax
import jax.numpy as jnp
from jax import lax
from jax.experimental import pallas as pl
from jax.experimental.pallas import tpu as pltpu
from jax.experimental.pallas import tpu_sc as plsc

N = 10000
E = 320000
C = 128
H = 8
D = 16
INV_SQRT_D = 1.0 / (D ** 0.5)

_BN = 1000

_NC = 2
_NS = 16
_NW = _NC * _NS
_EB = 16


def _proj_body(x_ref, wq_ref, wkv_ref, ws_ref, bq_ref, bkv_ref, bs_ref,
               q_ref, kv_ref, s_ref):
    xb = x_ref[...]
    q_ref[...] = jnp.dot(xb, wq_ref[...],
                         preferred_element_type=jnp.float32) + bq_ref[...]
    kv_ref[...] = jnp.dot(xb, wkv_ref[...],
                          preferred_element_type=jnp.float32) + bkv_ref[...]
    s_ref[...] = jnp.dot(xb, ws_ref[...],
                         preferred_element_type=jnp.float32) + bs_ref[...]


def _projections(x, Wq, bq, Wkv, bkv, Ws, bs):
    grid = (N // _BN,)
    full = lambda shape: pl.BlockSpec(shape, lambda i: (0, 0))
    return pl.pallas_call(
        _proj_body,
        grid=grid,
        in_specs=[
            pl.BlockSpec((_BN, C), lambda i: (i, 0)),
            full((C, C)), full((C, 2 * C)), full((C, C)),
            full((1, C)), full((1, 2 * C)), full((1, C)),
        ],
        out_specs=[
            pl.BlockSpec((_BN, C), lambda i: (i, 0)),
            pl.BlockSpec((_BN, 2 * C), lambda i: (i, 0)),
            pl.BlockSpec((_BN, C), lambda i: (i, 0)),
        ],
        out_shape=[
            jax.ShapeDtypeStruct((N, C), jnp.float32),
            jax.ShapeDtypeStruct((N, 2 * C), jnp.float32),
            jax.ShapeDtypeStruct((N, C), jnp.float32),
        ],
    )(x, Wq, Wkv, Ws, bq, bkv, bs)


_NB = N // _EB
_DR = 1280
_DB = _DR // _EB
_SB = 400
_IC = _SB // 16


def _edge_body(q_hbm, kv_hbm, src_hbm, dst_hbm,
               num_out, den_out,
               src_v, dst_v, qrows, kvrows, msg, den, semq, semkv,
               num_acc, den_acc):
    c = lax.axis_index("c")
    s = lax.axis_index("s")
    wid = s * _NC + c
    lane = lax.iota(jnp.int32, 16)
    zero16 = jnp.zeros((16,), jnp.float32)

    def zrow(r, carry):
        for h in range(H):
            msg[r, pl.ds(16 * h, 16)] = zero16
            den[r, pl.ds(16 * h, 16)] = zero16
        return carry

    lax.fori_loop(0, _EB, zrow, 0, unroll=False)

    def zblk(j, carry):
        bi = j * _NS + s

        @pl.when(bi < _NB)
        def _():
            pltpu.sync_copy(msg, num_acc.at[pl.ds(bi * _EB, _EB)])

        @pl.when(bi < _DB)
        def _():
            pltpu.sync_copy(msg, den_acc.at[pl.ds(bi * _EB, _EB)])

        return carry

    lax.fori_loop(0, (_NB + _NS - 1) // _NS, zblk, 0, unroll=False)
    plsc.subcore_barrier()

    epw = E // _NW
    base_w = wid * epw

    def super_body(sc_i, carry):
        base = base_w + sc_i * _SB
        pltpu.sync_copy(src_hbm.at[pl.ds(base, _SB)], src_v)
        pltpu.sync_copy(dst_hbm.at[pl.ds(base, _SB)], dst_v)

        def chunk_body(ci, carry2):
            dvec = dst_v[pl.ds(16 * ci, 16)]
            svec = src_v[pl.ds(16 * ci, 16)]
            cq = pltpu.async_copy(q_hbm.at[dvec], qrows, semq)
            ckv = pltpu.async_copy(kv_hbm.at[svec], kvrows, semkv)
            cq.wait()
            ckv.wait()
            seg = jnp.bitwise_and(dvec, 7)
            def head_body(h, carry3):
                hb = 16 * h

                def dot_body(d, acc_):
                    col = jnp.full((16,), hb + d, jnp.int32)
                    qv = plsc.load_gather(qrows, [lane, col])
                    kvv = plsc.load_gather(kvrows, [lane, col])
                    return acc_ + qv * kvv

                acc = lax.fori_loop(0, D, dot_body, jnp.zeros((16,), jnp.float32),
                                    unroll=4)
                w = jnp.exp(acc * INV_SQRT_D)

                def den_body(sb, carry4):
                    val = jnp.where(seg == sb, w, zero16)
                    cold = jnp.full((16,), 16 * sb + h, jnp.int32)
                    plsc.store_scatter(den, [lane, cold], val)
                    return carry4

                lax.fori_loop(0, 8, den_body, 0, unroll=4)

                def msg_body(d, carry5):
                    colv = jnp.full((16,), C + hb + d, jnp.int32)
                    vv = plsc.load_gather(kvrows, [lane, colv])
                    colm = jnp.full((16,), hb + d, jnp.int32)
                    plsc.store_scatter(msg, [lane, colm], w * vv)
                    return carry5

                lax.fori_loop(0, D, msg_body, 0, unroll=4)
                return carry3

            lax.fori_loop(0, H, head_body, 0, unroll=False)
            pltpu.sync_copy(msg, num_acc.at[dvec], add=True)
            pltpu.sync_copy(den, den_acc.at[lax.shift_right_logical(dvec, 3)],
                            add=True)
            return carry2

        lax.fori_loop(0, _IC, chunk_body, 0, unroll=False)
        return carry

    lax.fori_loop(0, epw // _SB, super_body, 0, unroll=False)
    plsc.subcore_barrier()

    def dblk(j, carry):
        bi = j * _NS + s

        @pl.when(bi < _NB)
        def _():
            pltpu.sync_copy(num_acc.at[pl.ds(bi * _EB, _EB)], msg)
            pltpu.sync_copy(msg, num_out.at[pl.ds(c * N + bi * _EB, _EB)])

        @pl.when(bi < _DB)
        def _():
            pltpu.sync_copy(den_acc.at[pl.ds(bi * _EB, _EB)], msg)
            pltpu.sync_copy(msg, den_out.at[pl.ds(c * _DR + bi * _EB, _EB)])

        return carry

    lax.fori_loop(0, (_NB + _NS - 1) // _NS, dblk, 0, unroll=False)


def _edge_stage_sc(q, kv, src, dst):
    call = pl.kernel(
        _edge_body,
        out_type=[
            jax.ShapeDtypeStruct((_NC * N, C), jnp.float32),
            jax.ShapeDtypeStruct((_NC * _DR, C), jnp.float32),
        ],
        mesh=plsc.VectorSubcoreMesh(core_axis_name="c", subcore_axis_name="s"),
        scratch_types=[
            pltpu.VMEM((_SB,), jnp.int32),
            pltpu.VMEM((_SB,), jnp.int32),
            pltpu.VMEM((16, C), jnp.float32),
            pltpu.VMEM((16, 2 * C), jnp.float32),
            pltpu.VMEM((16, C), jnp.float32),
            pltpu.VMEM((16, C), jnp.float32),
            pltpu.SemaphoreType.DMA,
            pltpu.SemaphoreType.DMA,
            pltpu.VMEM_SHARED((N, C), jnp.float32),
            pltpu.VMEM_SHARED((_DR, C), jnp.float32),
        ],
        compiler_params=pltpu.CompilerParams(needs_layout_passes=False),
    )
    num_flat, den_flat = call(q, kv, src, dst)
    den2 = den_flat.reshape(_NC, _DR * 8, 16)[:, :N, :]
    return num_flat.reshape(_NC, N, C), den2


def _post_body(x_ref, skip_ref, num_ref, den_ref, r_ref, w1_ref, b1_ref,
               w2_ref, b2_ref, g1_ref, be1_ref, g2_ref, be2_ref, out_ref):
    num = num_ref[0] + num_ref[1]
    den = den_ref[0] + den_ref[1]
    den_rep = jnp.dot(den, r_ref[...],
                      preferred_element_type=jnp.float32)
    agg = num / (den_rep + 1e-16)
    h0 = x_ref[...] + agg + skip_ref[...]
    mu = jnp.mean(h0, axis=-1, keepdims=True)
    var = jnp.mean((h0 - mu) ** 2, axis=-1, keepdims=True)
    h = (h0 - mu) / jnp.sqrt(var + 1e-5) * g1_ref[...] + be1_ref[...]
    a1 = jnp.maximum(jnp.dot(h, w1_ref[...],
                             preferred_element_type=jnp.float32)
                     + b1_ref[...], 0.0)
    ffn = jnp.dot(a1, w2_ref[...],
                  preferred_element_type=jnp.float32) + b2_ref[...]
    h2 = h + ffn
    mu2 = jnp.mean(h2, axis=-1, keepdims=True)
    var2 = jnp.mean((h2 - mu2) ** 2, axis=-1, keepdims=True)
    out_ref[...] = ((h2 - mu2) / jnp.sqrt(var2 + 1e-5) * g2_ref[...]
                    + be2_ref[...])


def _post_stage(x, skip, num2, den2, W1, b1, W2, b2, g1, be1, g2, be2):
    R = (jnp.arange(C)[None, :] // D == jnp.arange(16)[:, None]
         ).astype(jnp.float32)
    grid = (N // _BN,)
    full = lambda shape: pl.BlockSpec(shape, lambda i: (0, 0))
    row = lambda w: pl.BlockSpec((_BN, w), lambda i: (i, 0))
    return pl.pallas_call(
        _post_body,
        grid=grid,
        in_specs=[
            row(C), row(C),
            pl.BlockSpec((2, _BN, C), lambda i: (0, i, 0)),
            pl.BlockSpec((2, _BN, 16), lambda i: (0, i, 0)),
            full((16, C)),
            full((C, 4 * C)), full((1, 4 * C)),
            full((4 * C, C)), full((1, C)),
            full((1, C)), full((1, C)), full((1, C)), full((1, C)),
        ],
        out_specs=row(C),
        out_shape=jax.ShapeDtypeStruct((N, C), jnp.float32),
    )(x, skip, num2, den2, R, W1, b1, W2, b2, g1, be1, g2, be2)


def kernel(x, edge_index, Wq, bq, Wk, bk, Wv, bv, Ws, bs,
           ln1_g, ln1_b, W1, b1, W2, b2, ln2_g, ln2_b):
    src = edge_index[0].astype(jnp.int32)
    dst = edge_index[1].astype(jnp.int32)
    Wkv = jnp.concatenate([Wk, Wv], axis=1)
    bkv = jnp.concatenate([bk, bv])
    q, kv, skip = _projections(x, Wq, bq.reshape(1, C), Wkv,
                               bkv.reshape(1, 2 * C), Ws, bs.reshape(1, C))
    num2, den2 = _edge_stage_sc(q, kv, src, dst)
    return _post_stage(x, skip, num2, den2, W1, b1.reshape(1, 4 * C),
                       W2, b2.reshape(1, C), ln1_g.reshape(1, C),
                       ln1_b.reshape(1, C), ln2_g.reshape(1, C),
                       ln2_b.reshape(1, C))


# speedup vs baseline: 1.4563x; 1.4563x over previous
import functools

import jax
import jax.numpy as jnp
from jax import lax
from jax.experimental import pallas as pl
from jax.experimental.pallas import tpu as pltpu
from jax.experimental.pallas import tpu_sc as plsc

N = 10000
E = 320000
C = 128
H = 8
D = 16
INV_SQRT_D = 1.0 / (D ** 0.5)

_BN = 1000

_NC = 2
_NS = 16
_NW = _NC * _NS
_EB = 40


def _proj_body(x_ref, wq_ref, wkv_ref, ws_ref, bq_ref, bkv_ref, bs_ref,
               q_ref, kv_ref, s_ref):
    xb = x_ref[...]
    q_ref[...] = jnp.dot(xb, wq_ref[...],
                         preferred_element_type=jnp.float32) + bq_ref[...]
    kv_ref[...] = jnp.dot(xb, wkv_ref[...],
                          preferred_element_type=jnp.float32) + bkv_ref[...]
    s_ref[...] = jnp.dot(xb, ws_ref[...],
                         preferred_element_type=jnp.float32) + bs_ref[...]


def _projections(x, Wq, bq, Wkv, bkv, Ws, bs):
    grid = (N // _BN,)
    full = lambda shape: pl.BlockSpec(shape, lambda i: (0, 0))
    return pl.pallas_call(
        _proj_body,
        grid=grid,
        in_specs=[
            pl.BlockSpec((_BN, C), lambda i: (i, 0)),
            full((C, C)), full((C, 2 * C)), full((C, C)),
            full((1, C)), full((1, 2 * C)), full((1, C)),
        ],
        out_specs=[
            pl.BlockSpec((_BN, C), lambda i: (i, 0)),
            pl.BlockSpec((_BN, 2 * C), lambda i: (i, 0)),
            pl.BlockSpec((_BN, C), lambda i: (i, 0)),
        ],
        out_shape=[
            jax.ShapeDtypeStruct((N, C), jnp.float32),
            jax.ShapeDtypeStruct((N, 2 * C), jnp.float32),
            jax.ShapeDtypeStruct((N, C), jnp.float32),
        ],
    )(x, Wq, Wkv, Ws, bq, bkv, bs)


_NB = N // _EB
_DR = 1280
_DB = _DR // _EB


def _edge_body(q_hbm, kv_hbm, src_hbm, dst_hbm,
               num_out, den_out,
               src_v, dst_v, drow_v, qrows, kvrows, msg, den, semq, semkv,
               num_acc, den_acc):
    c = lax.axis_index("c")
    s = lax.axis_index("s")
    wid = s * _NC + c
    lane = lax.iota(jnp.int32, 16)
    zero16 = jnp.zeros((16,), jnp.float32)

    def zrow(r, carry):
        for h in range(H):
            msg[r, pl.ds(16 * h, 16)] = zero16
        return carry

    lax.fori_loop(0, _EB, zrow, 0, unroll=False)

    def zdrow(r, carry):
        for h in range(H):
            den[r, pl.ds(16 * h, 16)] = zero16
        return carry

    lax.fori_loop(0, _EB + 8, zdrow, 0, unroll=False)
    drow_v[pl.ds(32, 16)] = jnp.zeros((16,), jnp.int32)

    def zblk(j, carry):
        bi = j * _NS + s

        @pl.when(bi < _NB)
        def _():
            pltpu.sync_copy(msg, num_acc.at[pl.ds(bi * _EB, _EB)])

        @pl.when(bi < _DB)
        def _():
            pltpu.sync_copy(msg, den_acc.at[pl.ds(bi * _EB, _EB)])

        return carry

    lax.fori_loop(0, (_NB + _NS - 1) // _NS, zblk, 0, unroll=False)
    plsc.subcore_barrier()

    epw = E // _NW
    base_w = wid * epw
    xm = [jnp.bitwise_xor(lane, m) for m in (8, 4, 2, 1)]
    gdn = lax.GatherDimensionNumbers(offset_dims=(), collapsed_slice_dims=(0,),
                                     start_index_map=(0,))

    def take16(x, ix):
        return lax.gather(x, ix[:, None], gdn, (1,),
                          mode=lax.GatherScatterMode.PROMISE_IN_BOUNDS)

    def chunk_body(ci, carry):
        base = base_w + ci * _EB
        pltpu.sync_copy(src_hbm.at[pl.ds(base, _EB)], src_v)
        pltpu.sync_copy(dst_hbm.at[pl.ds(base, _EB)], dst_v)
        cq = pltpu.async_copy(q_hbm.at[dst_v], qrows, semq)
        ckv = pltpu.async_copy(kv_hbm.at[src_v], kvrows, semkv)
        cq.wait()
        ckv.wait()
        for g in (0, 1):
            dv = dst_v[pl.ds(16 * g, 16)]
            drow_v[pl.ds(16 * g, 16)] = lax.shift_right_logical(dv, 3)
        dv = dst_v[pl.ds(24, 16)]
        drow_v[pl.ds(24, 16)] = lax.shift_right_logical(dv, 3)

        def edge_body(e, carry2):
            wvecs = []
            for h in range(H):
                qh = qrows[e, pl.ds(16 * h, 16)]
                kh = kvrows[e, pl.ds(16 * h, 16)]
                p = qh * kh
                for ix in xm:
                    p = p + take16(p, ix)
                wh = jnp.exp(p * INV_SQRT_D)
                vh = kvrows[e, pl.ds(C + 16 * h, 16)]
                msg[e, pl.ds(16 * h, 16)] = wh * vh
                wvecs.append(wh)
            drow = jnp.zeros((16,), jnp.float32)
            for h in range(H):
                drow = jnp.where(lane == h, wvecs[h], drow)
            dvec = plsc.load_gather(dst_v, [jnp.full((16,), e, jnp.int32)])
            seg = jnp.bitwise_and(dvec, 7)
            for g in range(8):
                den[e, pl.ds(16 * g, 16)] = jnp.where(seg == g, drow, zero16)
            return carry2

        lax.fori_loop(0, _EB, edge_body, 0, unroll=False)
        pltpu.sync_copy(msg, num_acc.at[dst_v], add=True)
        pltpu.sync_copy(den, den_acc.at[drow_v], add=True)
        return carry

    lax.fori_loop(0, epw // _EB, chunk_body, 0, unroll=False)
    plsc.subcore_barrier()

    def dblk(j, carry):
        bi = j * _NS + s

        @pl.when(bi < _NB)
        def _():
            pltpu.sync_copy(num_acc.at[pl.ds(bi * _EB, _EB)], msg)
            pltpu.sync_copy(msg, num_out.at[pl.ds(c * N + bi * _EB, _EB)])

        @pl.when(bi < _DB)
        def _():
            pltpu.sync_copy(den_acc.at[pl.ds(bi * _EB, _EB)], msg)
            pltpu.sync_copy(msg, den_out.at[pl.ds(c * _DR + bi * _EB, _EB)])

        return carry

    lax.fori_loop(0, (_NB + _NS - 1) // _NS, dblk, 0, unroll=False)


def _edge_stage_sc(q, kv, src, dst):
    call = pl.kernel(
        _edge_body,
        out_type=[
            jax.ShapeDtypeStruct((_NC * N, C), jnp.float32),
            jax.ShapeDtypeStruct((_NC * _DR, C), jnp.float32),
        ],
        mesh=plsc.VectorSubcoreMesh(core_axis_name="c", subcore_axis_name="s"),
        scratch_types=[
            pltpu.VMEM((_EB,), jnp.int32),
            pltpu.VMEM((_EB,), jnp.int32),
            pltpu.VMEM((_EB + 8,), jnp.int32),
            pltpu.VMEM((_EB, C), jnp.float32),
            pltpu.VMEM((_EB, 2 * C), jnp.float32),
            pltpu.VMEM((_EB, C), jnp.float32),
            pltpu.VMEM((_EB + 8, C), jnp.float32),
            pltpu.SemaphoreType.DMA,
            pltpu.SemaphoreType.DMA,
            pltpu.VMEM_SHARED((N, C), jnp.float32),
            pltpu.VMEM_SHARED((_DR, C), jnp.float32),
        ],
        compiler_params=pltpu.CompilerParams(needs_layout_passes=False),
    )
    num_flat, den_flat = call(q, kv, src, dst)
    den2 = den_flat.reshape(_NC, _DR * 8, 16)[:, :N, :]
    return num_flat.reshape(_NC, N, C), den2


def _post_body(x_ref, skip_ref, num_ref, den_ref, r_ref, w1_ref, b1_ref,
               w2_ref, b2_ref, g1_ref, be1_ref, g2_ref, be2_ref, out_ref):
    num = num_ref[0] + num_ref[1]
    den = den_ref[0] + den_ref[1]
    den_rep = jnp.dot(den, r_ref[...],
                      preferred_element_type=jnp.float32)
    agg = num / (den_rep + 1e-16)
    h0 = x_ref[...] + agg + skip_ref[...]
    mu = jnp.mean(h0, axis=-1, keepdims=True)
    var = jnp.mean((h0 - mu) ** 2, axis=-1, keepdims=True)
    h = (h0 - mu) / jnp.sqrt(var + 1e-5) * g1_ref[...] + be1_ref[...]
    a1 = jnp.maximum(jnp.dot(h, w1_ref[...],
                             preferred_element_type=jnp.float32)
                     + b1_ref[...], 0.0)
    ffn = jnp.dot(a1, w2_ref[...],
                  preferred_element_type=jnp.float32) + b2_ref[...]
    h2 = h + ffn
    mu2 = jnp.mean(h2, axis=-1, keepdims=True)
    var2 = jnp.mean((h2 - mu2) ** 2, axis=-1, keepdims=True)
    out_ref[...] = ((h2 - mu2) / jnp.sqrt(var2 + 1e-5) * g2_ref[...]
                    + be2_ref[...])


def _post_stage(x, skip, num2, den2, W1, b1, W2, b2, g1, be1, g2, be2):
    R = (jnp.arange(C)[None, :] // D == jnp.arange(16)[:, None]
         ).astype(jnp.float32)
    grid = (N // _BN,)
    full = lambda shape: pl.BlockSpec(shape, lambda i: (0, 0))
    row = lambda w: pl.BlockSpec((_BN, w), lambda i: (i, 0))
    return pl.pallas_call(
        _post_body,
        grid=grid,
        in_specs=[
            row(C), row(C),
            pl.BlockSpec((2, _BN, C), lambda i: (0, i, 0)),
            pl.BlockSpec((2, _BN, 16), lambda i: (0, i, 0)),
            full((16, C)),
            full((C, 4 * C)), full((1, 4 * C)),
            full((4 * C, C)), full((1, C)),
            full((1, C)), full((1, C)), full((1, C)), full((1, C)),
        ],
        out_specs=row(C),
        out_shape=jax.ShapeDtypeStruct((N, C), jnp.float32),
    )(x, skip, num2, den2, R, W1, b1, W2, b2, g1, be1, g2, be2)


def kernel(x, edge_index, Wq, bq, Wk, bk, Wv, bv, Ws, bs,
           ln1_g, ln1_b, W1, b1, W2, b2, ln2_g, ln2_b):
    src = edge_index[0].astype(jnp.int32)
    dst = edge_index[1].astype(jnp.int32)
    Wkv = jnp.concatenate([Wk, Wv], axis=1)
    bkv = jnp.concatenate([bk, bv])
    q, kv, skip = _projections(x, Wq, bq.reshape(1, C), Wkv,
                               bkv.reshape(1, 2 * C), Ws, bs.reshape(1, C))
    num2, den2 = _edge_stage_sc(q, kv, src, dst)
    return _post_stage(x, skip, num2, den2, W1, b1.reshape(1, 4 * C),
                       W2, b2.reshape(1, C), ln1_g.reshape(1, C),
                       ln1_b.reshape(1, C), ln2_g.reshape(1, C),
                       ln2_b.reshape(1, C))


# speedup vs baseline: 5.3115x; 3.6473x over previous
import functools

import jax
import jax.numpy as jnp
from jax import lax
from jax.experimental import pallas as pl
from jax.experimental.pallas import tpu as pltpu
from jax.experimental.pallas import tpu_sc as plsc

N = 10000
E = 320000
C = 128
H = 8
D = 16
INV_SQRT_D = 1.0 / (D ** 0.5)

_BN = 1000

_NC = 2
_NS = 16
_NW = _NC * _NS
_EB = 40


def _proj_body(x_ref, wq_ref, wkv_ref, ws_ref, bq_ref, bkv_ref, bs_ref,
               q_ref, kv_ref, s_ref):
    xb = x_ref[...]
    q_ref[...] = jnp.dot(xb, wq_ref[...],
                         preferred_element_type=jnp.float32) + bq_ref[...]
    kv_ref[...] = jnp.dot(xb, wkv_ref[...],
                          preferred_element_type=jnp.float32) + bkv_ref[...]
    s_ref[...] = jnp.dot(xb, ws_ref[...],
                         preferred_element_type=jnp.float32) + bs_ref[...]


def _projections(x, Wq, bq, Wkv, bkv, Ws, bs):
    grid = (N // _BN,)
    full = lambda shape: pl.BlockSpec(shape, lambda i: (0, 0))
    return pl.pallas_call(
        _proj_body,
        grid=grid,
        in_specs=[
            pl.BlockSpec((_BN, C), lambda i: (i, 0)),
            full((C, C)), full((C, 2 * C)), full((C, C)),
            full((1, C)), full((1, 2 * C)), full((1, C)),
        ],
        out_specs=[
            pl.BlockSpec((_BN, C), lambda i: (i, 0)),
            pl.BlockSpec((_BN, 2 * C), lambda i: (i, 0)),
            pl.BlockSpec((_BN, C), lambda i: (i, 0)),
        ],
        out_shape=[
            jax.ShapeDtypeStruct((N, C), jnp.float32),
            jax.ShapeDtypeStruct((N, 2 * C), jnp.float32),
            jax.ShapeDtypeStruct((N, C), jnp.float32),
        ],
    )(x, Wq, Wkv, Ws, bq, bkv, bs)


_NB = N // _EB
_DR = 1280
_DB = _DR // _EB


def _edge_body(q_hbm, kv_hbm, src_hbm, dst_hbm,
               num_out, den_out,
               src_v, dst_v, drow_v, qrows, kvrows, msg, den, semq, semkv,
               num_acc, den_acc):
    c = lax.axis_index("c")
    s = lax.axis_index("s")
    wid = s * _NC + c
    lane = lax.iota(jnp.int32, 16)
    zero16 = jnp.zeros((16,), jnp.float32)

    def zrow(r, carry):
        for h in range(H):
            msg[r, pl.ds(16 * h, 16)] = zero16
        return carry

    lax.fori_loop(0, _EB, zrow, 0, unroll=False)

    def zdrow(r, carry):
        for h in range(H):
            den[r, pl.ds(16 * h, 16)] = zero16
        return carry

    lax.fori_loop(0, _EB + 8, zdrow, 0, unroll=False)
    drow_v[pl.ds(32, 16)] = jnp.zeros((16,), jnp.int32)

    def zblk(j, carry):
        bi = j * _NS + s

        @pl.when(bi < _NB)
        def _():
            pltpu.sync_copy(msg, num_acc.at[pl.ds(bi * _EB, _EB)])

        @pl.when(bi < _DB)
        def _():
            pltpu.sync_copy(msg, den_acc.at[pl.ds(bi * _EB, _EB)])

        return carry

    lax.fori_loop(0, (_NB + _NS - 1) // _NS, zblk, 0, unroll=False)
    plsc.subcore_barrier()

    epw = E // _NW
    base_w = wid * epw
    xm = [jnp.bitwise_xor(lane, m) for m in (8, 4, 2, 1)]
    gdn = lax.GatherDimensionNumbers(offset_dims=(), collapsed_slice_dims=(0,),
                                     start_index_map=(0,))

    def take16(x, ix):
        return lax.gather(x, ix[:, None], gdn, (1,),
                          mode=lax.GatherScatterMode.PROMISE_IN_BOUNDS)

    def chunk_body(ci, carry):
        base = base_w + ci * _EB
        pltpu.sync_copy(src_hbm.at[pl.ds(base, _EB)], src_v)
        pltpu.sync_copy(dst_hbm.at[pl.ds(base, _EB)], dst_v)
        cq = pltpu.async_copy(q_hbm.at[dst_v], qrows, semq)
        ckv = pltpu.async_copy(kv_hbm.at[src_v], kvrows, semkv)
        cq.wait()
        ckv.wait()
        for g in (0, 1):
            dv = dst_v[pl.ds(16 * g, 16)]
            drow_v[pl.ds(16 * g, 16)] = lax.shift_right_logical(dv, 3)
        dv = dst_v[pl.ds(24, 16)]
        drow_v[pl.ds(24, 16)] = lax.shift_right_logical(dv, 3)

        @functools.partial(plsc.parallel_loop, 0, _EB)
        def edge_body(e):
            wvecs = []
            for h in range(H):
                qh = qrows[e, pl.ds(16 * h, 16)]
                kh = kvrows[e, pl.ds(16 * h, 16)]
                p = qh * kh
                for ix in xm:
                    p = p + take16(p, ix)
                wh = jnp.exp(p * INV_SQRT_D)
                vh = kvrows[e, pl.ds(C + 16 * h, 16)]
                msg[e, pl.ds(16 * h, 16)] = wh * vh
                wvecs.append(wh)
            drow = jnp.zeros((16,), jnp.float32)
            for h in range(H):
                drow = jnp.where(lane == h, wvecs[h], drow)
            seg = jnp.bitwise_and(dst_v[e], 7)
            for g in range(8):
                den[e, pl.ds(16 * g, 16)] = jnp.where(seg == g, drow, zero16)
        pltpu.sync_copy(msg, num_acc.at[dst_v], add=True)
        pltpu.sync_copy(den, den_acc.at[drow_v], add=True)
        return carry

    lax.fori_loop(0, epw // _EB, chunk_body, 0, unroll=False)
    plsc.subcore_barrier()

    def dblk(j, carry):
        bi = j * _NS + s

        @pl.when(bi < _NB)
        def _():
            pltpu.sync_copy(num_acc.at[pl.ds(bi * _EB, _EB)], msg)
            pltpu.sync_copy(msg, num_out.at[pl.ds(c * N + bi * _EB, _EB)])

        @pl.when(bi < _DB)
        def _():
            pltpu.sync_copy(den_acc.at[pl.ds(bi * _EB, _EB)], msg)
            pltpu.sync_copy(msg, den_out.at[pl.ds(c * _DR + bi * _EB, _EB)])

        return carry

    lax.fori_loop(0, (_NB + _NS - 1) // _NS, dblk, 0, unroll=False)


def _edge_stage_sc(q, kv, src, dst):
    call = pl.kernel(
        _edge_body,
        out_type=[
            jax.ShapeDtypeStruct((_NC * N, C), jnp.float32),
            jax.ShapeDtypeStruct((_NC * _DR, C), jnp.float32),
        ],
        mesh=plsc.VectorSubcoreMesh(core_axis_name="c", subcore_axis_name="s"),
        scratch_types=[
            pltpu.VMEM((_EB,), jnp.int32),
            pltpu.VMEM((_EB,), jnp.int32),
            pltpu.VMEM((_EB + 8,), jnp.int32),
            pltpu.VMEM((_EB, C), jnp.float32),
            pltpu.VMEM((_EB, 2 * C), jnp.float32),
            pltpu.VMEM((_EB, C), jnp.float32),
            pltpu.VMEM((_EB + 8, C), jnp.float32),
            pltpu.SemaphoreType.DMA,
            pltpu.SemaphoreType.DMA,
            pltpu.VMEM_SHARED((N, C), jnp.float32),
            pltpu.VMEM_SHARED((_DR, C), jnp.float32),
        ],
        compiler_params=pltpu.CompilerParams(needs_layout_passes=False),
    )
    num_flat, den_flat = call(q, kv, src, dst)
    den2 = den_flat.reshape(_NC, _DR * 8, 16)[:, :N, :]
    return num_flat.reshape(_NC, N, C), den2


def _post_body(x_ref, skip_ref, num_ref, den_ref, r_ref, w1_ref, b1_ref,
               w2_ref, b2_ref, g1_ref, be1_ref, g2_ref, be2_ref, out_ref):
    num = num_ref[0] + num_ref[1]
    den = den_ref[0] + den_ref[1]
    den_rep = jnp.dot(den, r_ref[...],
                      preferred_element_type=jnp.float32)
    agg = num / (den_rep + 1e-16)
    h0 = x_ref[...] + agg + skip_ref[...]
    mu = jnp.mean(h0, axis=-1, keepdims=True)
    var = jnp.mean((h0 - mu) ** 2, axis=-1, keepdims=True)
    h = (h0 - mu) / jnp.sqrt(var + 1e-5) * g1_ref[...] + be1_ref[...]
    a1 = jnp.maximum(jnp.dot(h, w1_ref[...],
                             preferred_element_type=jnp.float32)
                     + b1_ref[...], 0.0)
    ffn = jnp.dot(a1, w2_ref[...],
                  preferred_element_type=jnp.float32) + b2_ref[...]
    h2 = h + ffn
    mu2 = jnp.mean(h2, axis=-1, keepdims=True)
    var2 = jnp.mean((h2 - mu2) ** 2, axis=-1, keepdims=True)
    out_ref[...] = ((h2 - mu2) / jnp.sqrt(var2 + 1e-5) * g2_ref[...]
                    + be2_ref[...])


def _post_stage(x, skip, num2, den2, W1, b1, W2, b2, g1, be1, g2, be2):
    R = (jnp.arange(C)[None, :] // D == jnp.arange(16)[:, None]
         ).astype(jnp.float32)
    grid = (N // _BN,)
    full = lambda shape: pl.BlockSpec(shape, lambda i: (0, 0))
    row = lambda w: pl.BlockSpec((_BN, w), lambda i: (i, 0))
    return pl.pallas_call(
        _post_body,
        grid=grid,
        in_specs=[
            row(C), row(C),
            pl.BlockSpec((2, _BN, C), lambda i: (0, i, 0)),
            pl.BlockSpec((2, _BN, 16), lambda i: (0, i, 0)),
            full((16, C)),
            full((C, 4 * C)), full((1, 4 * C)),
            full((4 * C, C)), full((1, C)),
            full((1, C)), full((1, C)), full((1, C)), full((1, C)),
        ],
        out_specs=row(C),
        out_shape=jax.ShapeDtypeStruct((N, C), jnp.float32),
    )(x, skip, num2, den2, R, W1, b1, W2, b2, g1, be1, g2, be2)


def kernel(x, edge_index, Wq, bq, Wk, bk, Wv, bv, Ws, bs,
           ln1_g, ln1_b, W1, b1, W2, b2, ln2_g, ln2_b):
    src = edge_index[0].astype(jnp.int32)
    dst = edge_index[1].astype(jnp.int32)
    Wkv = jnp.concatenate([Wk, Wv], axis=1)
    bkv = jnp.concatenate([bk, bv])
    q, kv, skip = _projections(x, Wq, bq.reshape(1, C), Wkv,
                               bkv.reshape(1, 2 * C), Ws, bs.reshape(1, C))
    num2, den2 = _edge_stage_sc(q, kv, src, dst)
    return _post_stage(x, skip, num2, den2, W1, b1.reshape(1, 4 * C),
                       W2, b2.reshape(1, C), ln1_g.reshape(1, C),
                       ln1_b.reshape(1, C), ln2_g.reshape(1, C),
                       ln2_b.reshape(1, C))
